# baseline (speedup 1.0000x reference)
import jax
import jax.numpy as jnp
from jax.experimental import pallas as pl

_N = 4194304
_GROUPS = _N // 512
_BLOCK_GROUPS = 2048


def _body(u_ref, o_ref):
    v = u_ref[...]
    sub = jax.lax.broadcasted_iota(jnp.int32, v.shape, dimension=1)
    ev = (sub & 1) == 0
    ge = jnp.where(ev, jnp.float32(0.05), jnp.float32(-0.05))
    nge = jnp.where(ev, jnp.float32(-0.05), jnp.float32(0.05))
    cb = jnp.where(ev, jnp.float32(-0.02), jnp.float32(0.02))
    p = jnp.where(ev, jnp.roll(v, -1, axis=1), jnp.roll(v, 1, axis=1))
    g = jnp.where(p > 0.0, ge, nge)
    av = jnp.abs(v)
    ap = jnp.abs(p)
    yq = v - g * (av - jnp.float32(1.0))
    m = jnp.maximum(av, ap)
    ub = jnp.float32(1.5)
    inside = m < ub
    bd = m > ub
    ybd = jnp.float32(0.9505) * v + cb * p
    good = jnp.where(ev, jnp.maximum(av, p), jnp.maximum(ap, v)) > 0.0
    y = jnp.where(inside & good, yq, jnp.float32(0.0))
    o_ref[...] = jnp.where(bd, ybd, y)


def kernel(z):
    u = (z.reshape(_GROUPS, 4, 128, 2)
         .transpose(0, 1, 3, 2)
         .reshape(_GROUPS, 8, 128))
    spec = pl.BlockSpec((_BLOCK_GROUPS, 8, 128), lambda i: (i, 0, 0))
    y = pl.pallas_call(
        _body,
        out_shape=jax.ShapeDtypeStruct((_GROUPS, 8, 128), jnp.float32),
        grid=(_GROUPS // _BLOCK_GROUPS,),
        in_specs=[spec],
        out_specs=spec,
    )(u)
    return (y.reshape(_GROUPS, 4, 2, 128)
            .transpose(0, 1, 3, 2)
            .reshape(_N, 2))


# speedup vs baseline: 1.0106x; 1.0106x over previous
import jax
import jax.numpy as jnp
from jax.experimental import pallas as pl
from jax.experimental.pallas import tpu as pltpu

_N = 4194304
_GROUPS = _N // 512
_BLOCK_GROUPS = 1024


def _body(u_ref, o_ref):
    v = u_ref[...]
    sub = jax.lax.broadcasted_iota(jnp.int32, v.shape, dimension=1)
    ev = (sub & 1) == 0
    ge = jnp.where(ev, jnp.float32(0.05), jnp.float32(-0.05))
    nge = jnp.where(ev, jnp.float32(-0.05), jnp.float32(0.05))
    cb = jnp.where(ev, jnp.float32(-0.02), jnp.float32(0.02))
    p = jnp.where(ev, jnp.roll(v, -1, axis=1), jnp.roll(v, 1, axis=1))
    g = jnp.where(p > 0.0, ge, nge)
    av = jnp.abs(v)
    ap = jnp.abs(p)
    yq = v - g * (av - jnp.float32(1.0))
    m = jnp.maximum(av, ap)
    ub = jnp.float32(1.5)
    inside = m < ub
    bd = m > ub
    ybd = jnp.float32(0.9505) * v + cb * p
    good = jnp.where(ev, jnp.maximum(av, p), jnp.maximum(ap, v)) > 0.0
    y = jnp.where(inside & good, yq, jnp.float32(0.0))
    o_ref[...] = jnp.where(bd, ybd, y)


def kernel(z):
    u = (z.reshape(_GROUPS, 4, 128, 2)
         .transpose(0, 1, 3, 2)
         .reshape(_GROUPS, 8, 128))
    spec = pl.BlockSpec((_BLOCK_GROUPS, 8, 128), lambda i: (i, 0, 0))
    y = pl.pallas_call(
        _body,
        out_shape=jax.ShapeDtypeStruct((_GROUPS, 8, 128), jnp.float32),
        grid=(_GROUPS // _BLOCK_GROUPS,),
        in_specs=[spec],
        out_specs=spec,
        compiler_params=pltpu.CompilerParams(
            dimension_semantics=("parallel",)),
    )(u)
    return (y.reshape(_GROUPS, 4, 2, 128)
            .transpose(0, 1, 3, 2)
            .reshape(_N, 2))


# speedup vs baseline: 1.0119x; 1.0013x over previous
import jax
import jax.numpy as jnp
from jax.experimental import pallas as pl

_N = 4194304
_GROUPS = _N // 512
_BLOCK_GROUPS = 1024


def _body(u_ref, o_ref):
    v = u_ref[...]
    sub = jax.lax.broadcasted_iota(jnp.int32, v.shape, dimension=1)
    ev = (sub & 1) == 0
    ge = jnp.where(ev, jnp.float32(0.05), jnp.float32(-0.05))
    nge = jnp.where(ev, jnp.float32(-0.05), jnp.float32(0.05))
    cb = jnp.where(ev, jnp.float32(-0.02), jnp.float32(0.02))
    p = jnp.where(ev, jnp.roll(v, -1, axis=1), jnp.roll(v, 1, axis=1))
    g = jnp.where(p > 0.0, ge, nge)
    av = jnp.abs(v)
    ap = jnp.abs(p)
    yq = v - g * (av - jnp.float32(1.0))
    m = jnp.maximum(av, ap)
    ub = jnp.float32(1.5)
    inside = m < ub
    bd = m > ub
    ybd = jnp.float32(0.9505) * v + cb * p
    good = jnp.where(ev, jnp.maximum(av, p), jnp.maximum(ap, v)) > 0.0
    y = jnp.where(inside & good, yq, jnp.float32(0.0))
    o_ref[...] = jnp.where(bd, ybd, y)


def kernel(z):
    u = (z.reshape(_GROUPS, 4, 128, 2)
         .transpose(0, 1, 3, 2)
         .reshape(_GROUPS, 8, 128))
    spec = pl.BlockSpec((_BLOCK_GROUPS, 8, 128), lambda i: (i, 0, 0))
    y = pl.pallas_call(
        _body,
        out_shape=jax.ShapeDtypeStruct((_GROUPS, 8, 128), jnp.float32),
        grid=(_GROUPS // _BLOCK_GROUPS,),
        in_specs=[spec],
        out_specs=spec,
    )(u)
    return (y.reshape(_GROUPS, 4, 2, 128)
            .transpose(0, 1, 3, 2)
            .reshape(_N, 2))
